# Initial kernel scaffold; baseline (speedup 1.0000x reference)
import jax
import jax.numpy as jnp
from jax.experimental import pallas as pl


def kernel(x, edge_index, W1, as1, ad1, b1, W2, as2, ad2, b2, W3, as3, ad3, b3):
    raise NotImplementedError("write your pallas kernel here")


# speedup vs baseline: 12.6753x; 12.6753x over previous
import functools

import jax
import jax.numpy as jnp
from jax import lax
from jax.experimental import pallas as pl
from jax.experimental.pallas import tpu as pltpu
from jax.experimental.pallas import tpu_sc as plsc

N = 10000
E = 320000
NC = 2
NS = 16
LANES = 16
CH = 128
EA = E // NS
EB = E // (NC * NS)
ROWS_MAIN = 640
ROWS_LAST = N - (NS - 1) * ROWS_MAIN
NEG_SLOPE = 0.2
EPS = 1e-16
BR = 1000


def _leaky(v):
    return jnp.where(v >= 0.0, v, NEG_SLOPE * v)


def _sc_edge_body(D, h_hbm, as_hbm, ad_hbm, src_hbm, dst_hbm,
                  out_hbm, rec_hbm,
                  rcomp_v, asb, adb, reb, coeff_v, wv,
                  sidx, didx, ridx, rows,
                  sidx_t, didx_t, ridx_t, rows_t, asb_t, adb_t, reb_t,
                  sidx_a, didx_a, asb_a, adb_a,
                  gsem, sp_dn, sp_out):
    cid = lax.axis_index("c")
    sid = lax.axis_index("s")
    wid = sid * NC + cid
    KD = D // LANES
    recbase = cid * N

    def _dn_init(rbase, nchunks, crows):
        for k in range(nchunks):
            kb = rbase + k * crows
            pltpu.sync_copy(as_hbm.at[pl.ds(kb, crows)], asb.at[pl.ds(0, crows)])
            pltpu.sync_copy(ad_hbm.at[pl.ds(kb, crows)], adb.at[pl.ds(0, crows)])
            for g in range(crows // LANES):
                o = g * LANES
                a = asb[pl.ds(o, LANES)] + adb[pl.ds(o, LANES)]
                wv[pl.ds(o, LANES)] = jnp.exp(_leaky(a))
            pltpu.sync_copy(wv.at[pl.ds(0, crows)], sp_dn.at[pl.ds(kb, crows)])

    @pl.when(sid < NS - 1)
    def _():
        _dn_init(sid * ROWS_MAIN, 5, 128)

    @pl.when(sid == NS - 1)
    def _():
        _dn_init((NS - 1) * ROWS_MAIN, 5, ROWS_LAST // 5)

    plsc.subcore_barrier()

    ebase = sid * EA
    nfa = EA // CH
    rema = EA - nfa * CH

    def _pa_body(ci, _):
        cb = ebase + ci * CH
        pltpu.sync_copy(src_hbm.at[pl.ds(cb, CH)], sidx)
        pltpu.sync_copy(dst_hbm.at[pl.ds(cb, CH)], didx)
        pltpu.sync_copy(as_hbm.at[sidx], asb)
        pltpu.sync_copy(ad_hbm.at[didx], adb)
        for g in range(CH // LANES):
            o = g * LANES
            a = asb[pl.ds(o, LANES)] + adb[pl.ds(o, LANES)]
            wv[pl.ds(o, LANES)] = jnp.exp(_leaky(a))
        pltpu.sync_copy(wv, sp_dn.at[didx], add=True)
        return 0

    lax.fori_loop(0, nfa, _pa_body, 0)

    tb = ebase + nfa * CH
    pltpu.sync_copy(src_hbm.at[pl.ds(tb, rema)], sidx_a)
    pltpu.sync_copy(dst_hbm.at[pl.ds(tb, rema)], didx_a)
    pltpu.sync_copy(as_hbm.at[sidx_a], asb_a)
    pltpu.sync_copy(ad_hbm.at[didx_a], adb_a)
    for g in range(rema // LANES):
        o = g * LANES
        a = asb_a[pl.ds(o, LANES)] + adb_a[pl.ds(o, LANES)]
        wv[pl.ds(o, LANES)] = jnp.exp(_leaky(a))
    pltpu.sync_copy(wv.at[pl.ds(0, rema)], sp_dn.at[didx_a], add=True)

    plsc.subcore_barrier()

    def _recip(rbase, nchunks, crows):
        for k in range(nchunks):
            kb = rbase + k * crows
            pltpu.sync_copy(sp_dn.at[pl.ds(kb, crows)], wv.at[pl.ds(0, crows)])
            for g in range(crows // LANES):
                d16 = wv[pl.ds(g * LANES, LANES)]
                rcomp_v[pl.ds(k * crows + g * LANES, LANES)] = 1.0 / (d16 + EPS)
        nr = nchunks * crows
        pltpu.sync_copy(rcomp_v.at[pl.ds(0, nr)],
                        rec_hbm.at[pl.ds(recbase + rbase, nr)])

    @pl.when(sid < NS - 1)
    def _():
        _recip(sid * ROWS_MAIN, 5, 128)

    @pl.when(sid == NS - 1)
    def _():
        _recip((NS - 1) * ROWS_MAIN, 5, ROWS_LAST // 5)

    plsc.subcore_barrier()

    def _scale_rows(rref, nrows):
        def _s(g, _):
            c16 = coeff_v[pl.ds(g * LANES, LANES)]
            for jj in range(LANES):
                c = c16[jj]
                row = g * LANES + jj
                for kk in range(KD):
                    rref[row, pl.ds(kk * LANES, LANES)] = (
                        rref[row, pl.ds(kk * LANES, LANES)] * c)
            return 0
        lax.fori_loop(0, nrows // LANES, _s, 0)

    def _selfloop(rbase, nchunks, crows):
        for k in range(nchunks):
            kb = rbase + k * crows
            pltpu.sync_copy(h_hbm.at[pl.ds(kb, crows), :],
                            rows.at[pl.ds(0, crows)])
            pltpu.sync_copy(as_hbm.at[pl.ds(kb, crows)], asb.at[pl.ds(0, crows)])
            pltpu.sync_copy(ad_hbm.at[pl.ds(kb, crows)], adb.at[pl.ds(0, crows)])
            pltpu.sync_copy(rec_hbm.at[pl.ds(recbase + kb, crows)],
                            reb.at[pl.ds(0, crows)])
            for g in range(crows // LANES):
                o = g * LANES
                a = asb[pl.ds(o, LANES)] + adb[pl.ds(o, LANES)]
                w = jnp.exp(_leaky(a))
                coeff_v[pl.ds(o, LANES)] = w * reb[pl.ds(o, LANES)]
            _scale_rows(rows, crows)
            pltpu.sync_copy(rows.at[pl.ds(0, crows)], sp_out.at[pl.ds(kb, crows)])

    def _zero_out(rbase, nchunks, crows):
        def _z(j, _):
            for kk in range(KD):
                rows[j, pl.ds(kk * LANES, LANES)] = jnp.zeros(
                    (LANES,), jnp.float32)
            return 0
        lax.fori_loop(0, CH, _z, 0)
        for k in range(nchunks):
            kb = rbase + k * crows
            pltpu.sync_copy(rows.at[pl.ds(0, crows)], sp_out.at[pl.ds(kb, crows)])

    @pl.when(jnp.logical_and(cid == 0, sid < NS - 1))
    def _():
        _selfloop(sid * ROWS_MAIN, 5, 128)

    @pl.when(jnp.logical_and(cid == 0, sid == NS - 1))
    def _():
        _selfloop((NS - 1) * ROWS_MAIN, 5, ROWS_LAST // 5)

    @pl.when(jnp.logical_and(cid == 1, sid < NS - 1))
    def _():
        _zero_out(sid * ROWS_MAIN, 5, 128)

    @pl.when(jnp.logical_and(cid == 1, sid == NS - 1))
    def _():
        _zero_out((NS - 1) * ROWS_MAIN, 5, ROWS_LAST // 5)

    plsc.subcore_barrier()

    wbase = wid * EB
    nfb = EB // CH
    remb = EB - nfb * CH

    def _pb_body(ci, _):
        cb = wbase + ci * CH
        pltpu.sync_copy(src_hbm.at[pl.ds(cb, CH)], sidx)
        pltpu.sync_copy(dst_hbm.at[pl.ds(cb, CH)], didx)
        cp = pltpu.async_copy(h_hbm.at[sidx], rows, gsem)
        for g in range(CH // LANES):
            o = g * LANES
            ridx[pl.ds(o, LANES)] = didx[pl.ds(o, LANES)] + recbase
        pltpu.sync_copy(as_hbm.at[sidx], asb)
        pltpu.sync_copy(ad_hbm.at[didx], adb)
        pltpu.sync_copy(rec_hbm.at[ridx], reb)
        for g in range(CH // LANES):
            o = g * LANES
            a = asb[pl.ds(o, LANES)] + adb[pl.ds(o, LANES)]
            w = jnp.exp(_leaky(a))
            coeff_v[pl.ds(o, LANES)] = w * reb[pl.ds(o, LANES)]
        cp.wait()
        _scale_rows(rows, CH)
        pltpu.sync_copy(rows, sp_out.at[didx], add=True)
        return 0

    lax.fori_loop(0, nfb, _pb_body, 0)

    tb2 = wbase + nfb * CH
    pltpu.sync_copy(src_hbm.at[pl.ds(tb2, remb)], sidx_t)
    pltpu.sync_copy(dst_hbm.at[pl.ds(tb2, remb)], didx_t)
    cp = pltpu.async_copy(h_hbm.at[sidx_t], rows_t, gsem)
    ridx_t[...] = didx_t[...] + recbase
    pltpu.sync_copy(as_hbm.at[sidx_t], asb_t)
    pltpu.sync_copy(ad_hbm.at[didx_t], adb_t)
    pltpu.sync_copy(rec_hbm.at[ridx_t], reb_t)
    a = asb_t[...] + adb_t[...]
    w = jnp.exp(_leaky(a))
    coeff_v[pl.ds(0, LANES)] = w * reb_t[...]
    cp.wait()
    _scale_rows(rows_t, remb)
    pltpu.sync_copy(rows_t, sp_out.at[didx_t], add=True)

    plsc.subcore_barrier()

    def _export(rbase, nchunks, crows):
        for k in range(nchunks):
            kb = rbase + k * crows
            pltpu.sync_copy(sp_out.at[pl.ds(kb, crows)], rows.at[pl.ds(0, crows)])
            pltpu.sync_copy(rows.at[pl.ds(0, crows)],
                            out_hbm.at[cid, pl.ds(kb, crows), :])

    @pl.when(sid < NS - 1)
    def _():
        _export(sid * ROWS_MAIN, 5, 128)

    @pl.when(sid == NS - 1)
    def _():
        _export((NS - 1) * ROWS_MAIN, 5, ROWS_LAST // 5)


@functools.lru_cache(maxsize=None)
def _make_sc_edge(D):
    mesh = plsc.VectorSubcoreMesh(core_axis_name="c", subcore_axis_name="s",
                                  num_cores=NC, num_subcores=NS)
    scratch = [
        pltpu.VMEM((ROWS_MAIN,), jnp.float32),
        pltpu.VMEM((CH,), jnp.float32),
        pltpu.VMEM((CH,), jnp.float32),
        pltpu.VMEM((CH,), jnp.float32),
        pltpu.VMEM((CH,), jnp.float32),
        pltpu.VMEM((CH,), jnp.float32),
        pltpu.VMEM((CH,), jnp.int32),
        pltpu.VMEM((CH,), jnp.int32),
        pltpu.VMEM((CH,), jnp.int32),
        pltpu.VMEM((CH, D), jnp.float32),
        pltpu.VMEM((LANES,), jnp.int32),
        pltpu.VMEM((LANES,), jnp.int32),
        pltpu.VMEM((LANES,), jnp.int32),
        pltpu.VMEM((LANES, D), jnp.float32),
        pltpu.VMEM((LANES,), jnp.float32),
        pltpu.VMEM((LANES,), jnp.float32),
        pltpu.VMEM((LANES,), jnp.float32),
        pltpu.VMEM((32,), jnp.int32),
        pltpu.VMEM((32,), jnp.int32),
        pltpu.VMEM((32,), jnp.float32),
        pltpu.VMEM((32,), jnp.float32),
        pltpu.SemaphoreType.DMA,
        pltpu.VMEM_SHARED((N,), jnp.float32),
        pltpu.VMEM_SHARED((N, D), jnp.float32),
    ]
    return pl.kernel(
        functools.partial(_sc_edge_body, D),
        out_type=(jax.ShapeDtypeStruct((NC, N, D), jnp.float32),
                  jax.ShapeDtypeStruct((NC * N,), jnp.float32)),
        mesh=mesh,
        scratch_types=scratch,
        compiler_params=pltpu.CompilerParams(needs_layout_passes=False),
    )


def _tc_pre_body(x_ref, w_ref, as_ref, ad_ref, h_ref, asv_ref, adv_ref):
    h = jnp.dot(x_ref[...], w_ref[...], preferred_element_type=jnp.float32)
    h_ref[...] = h
    asv_ref[...] = jnp.sum(h * as_ref[...], axis=1).reshape(1, 1, -1)
    adv_ref[...] = jnp.sum(h * ad_ref[...], axis=1).reshape(1, 1, -1)


def _tc_pre(x, W, a_s, a_d):
    Cin, Cout = W.shape
    f = pl.pallas_call(
        _tc_pre_body,
        grid=(N // BR,),
        in_specs=[
            pl.BlockSpec((BR, Cin), lambda i: (i, 0)),
            pl.BlockSpec((Cin, Cout), lambda i: (0, 0)),
            pl.BlockSpec((1, Cout), lambda i: (0, 0)),
            pl.BlockSpec((1, Cout), lambda i: (0, 0)),
        ],
        out_specs=[
            pl.BlockSpec((BR, Cout), lambda i: (i, 0)),
            pl.BlockSpec((1, 1, BR), lambda i: (i, 0, 0)),
            pl.BlockSpec((1, 1, BR), lambda i: (i, 0, 0)),
        ],
        out_shape=[
            jax.ShapeDtypeStruct((N, Cout), jnp.float32),
            jax.ShapeDtypeStruct((N // BR, 1, BR), jnp.float32),
            jax.ShapeDtypeStruct((N // BR, 1, BR), jnp.float32),
        ],
    )
    return f(x, W, a_s, a_d)


def _tc_mid_body(p_ref, b_ref, w_ref, as_ref, ad_ref, h_ref, asv_ref, adv_ref):
    o = jnp.maximum(p_ref[0] + p_ref[1] + b_ref[...], 0.0)
    h = jnp.dot(o, w_ref[...], preferred_element_type=jnp.float32)
    h_ref[...] = h
    asv_ref[...] = jnp.sum(h * as_ref[...], axis=1).reshape(1, 1, -1)
    adv_ref[...] = jnp.sum(h * ad_ref[...], axis=1).reshape(1, 1, -1)


def _tc_mid(p, b, W, a_s, a_d):
    Cin, Cout = W.shape
    f = pl.pallas_call(
        _tc_mid_body,
        grid=(N // BR,),
        in_specs=[
            pl.BlockSpec((NC, BR, Cin), lambda i: (0, i, 0)),
            pl.BlockSpec((1, Cin), lambda i: (0, 0)),
            pl.BlockSpec((Cin, Cout), lambda i: (0, 0)),
            pl.BlockSpec((1, Cout), lambda i: (0, 0)),
            pl.BlockSpec((1, Cout), lambda i: (0, 0)),
        ],
        out_specs=[
            pl.BlockSpec((BR, Cout), lambda i: (i, 0)),
            pl.BlockSpec((1, 1, BR), lambda i: (i, 0, 0)),
            pl.BlockSpec((1, 1, BR), lambda i: (i, 0, 0)),
        ],
        out_shape=[
            jax.ShapeDtypeStruct((N, Cout), jnp.float32),
            jax.ShapeDtypeStruct((N // BR, 1, BR), jnp.float32),
            jax.ShapeDtypeStruct((N // BR, 1, BR), jnp.float32),
        ],
    )
    return f(p, b.reshape(1, -1), W, a_s, a_d)


def _tc_fin_body(C, p_ref, b_ref, o_ref):
    o_ref[...] = p_ref[0, :, :C] + p_ref[1, :, :C] + b_ref[...]


def _tc_fin(p, b):
    C = b.size
    Cp = p.shape[-1]
    f = pl.pallas_call(
        functools.partial(_tc_fin_body, C),
        grid=(N // BR,),
        in_specs=[
            pl.BlockSpec((NC, BR, Cp), lambda i: (0, i, 0)),
            pl.BlockSpec((1, C), lambda i: (0, 0)),
        ],
        out_specs=pl.BlockSpec((BR, C), lambda i: (i, 0)),
        out_shape=jax.ShapeDtypeStruct((N, C), jnp.float32),
    )
    return f(p, b.reshape(1, -1))


def kernel(x, edge_index, W1, as1, ad1, b1, W2, as2, ad2, b2,
           W3, as3, ad3, b3):
    src = edge_index[0].astype(jnp.int32)
    dst = edge_index[1].astype(jnp.int32)

    sc128 = _make_sc_edge(128)

    pad = 128 - W3.shape[1]
    W3p = jnp.pad(W3, ((0, 0), (0, pad)))
    as3p = jnp.pad(as3, ((0, 0), (0, pad)))
    ad3p = jnp.pad(ad3, ((0, 0), (0, pad)))

    h, asv, adv = _tc_pre(x, W1, as1, ad1)
    p, _ = sc128(h, asv.reshape(N), adv.reshape(N), src, dst)
    h, asv, adv = _tc_mid(p, b1, W2, as2, ad2)
    p, _ = sc128(h, asv.reshape(N), adv.reshape(N), src, dst)
    h, asv, adv = _tc_mid(p, b2, W3p, as3p, ad3p)
    p, _ = sc128(h, asv.reshape(N), adv.reshape(N), src, dst)
    return _tc_fin(p, b3)


# speedup vs baseline: 52.3924x; 4.1334x over previous
import functools

import jax
import jax.numpy as jnp
from jax import lax
from jax.experimental import pallas as pl
from jax.experimental.pallas import tpu as pltpu
from jax.experimental.pallas import tpu_sc as plsc

N = 10000
E = 320000
NC = 2
NS = 16
LANES = 16
D = 128
CH = 80
NSUB = E // (NC * NS * CH)
ROWS_MAIN = 640
ROWS_LAST = N - (NS - 1) * ROWS_MAIN
NEG_SLOPE = 0.2
EPS = 1e-16
BR = 1000
KD = D // LANES


def _leaky(v):
    return jnp.where(v >= 0.0, v, NEG_SLOPE * v)


def _sc_edge_body(h_hbm, as_hbm, ad_hbm, src2_hbm, dst2_hbm,
                  out_hbm, dn_hbm,
                  sidx_all, didx_all, rcomp_v,
                  asb0, asb1, adb0, adb1, wv0, wv1, rows0, rows1, zv,
                  ga0, ga1, gad0, gad1, gr0, gr1, wdn0, wdn1, wrow0, wrow1,
                  sp_dn, sp_out):
    cid = lax.axis_index("c")
    sid = lax.axis_index("s")
    wid = sid * NC + cid
    dnbase = cid * N

    asb = [asb0, asb1]
    adb = [adb0, adb1]
    wv = [wv0, wv1]
    rows = [rows0, rows1]
    ga = [ga0, ga1]
    gad = [gad0, gad1]
    gr = [gr0, gr1]
    wdn = [wdn0, wdn1]
    wrow = [wrow0, wrow1]

    pltpu.sync_copy(src2_hbm.at[pl.ds(wid * NSUB * CH, NSUB * CH)], sidx_all)
    pltpu.sync_copy(dst2_hbm.at[wid], didx_all)

    for g in range(CH // LANES):
        zv[pl.ds(g * LANES, LANES)] = jnp.zeros((LANES,), jnp.float32)

    def _zero_rows(rref):
        def _z(j, _):
            for kk in range(KD):
                rref[j, pl.ds(kk * LANES, LANES)] = jnp.zeros(
                    (LANES,), jnp.float32)
            return 0
        lax.fori_loop(0, CH, _z, 0)

    def _scale_rows(rref, cref):
        def _s(g, _):
            c16 = cref[pl.ds(g * LANES, LANES)]
            for jj in range(LANES):
                c = c16[jj]
                row = g * LANES + jj
                for kk in range(KD):
                    rref[row, pl.ds(kk * LANES, LANES)] = (
                        rref[row, pl.ds(kk * LANES, LANES)] * c)
            return 0
        lax.fori_loop(0, CH // LANES, _s, 0)

    def _init_self(rbase, nchunks):
        for k in range(nchunks):
            kb = rbase + k * CH
            pltpu.sync_copy(h_hbm.at[pl.ds(kb, CH), :], rows0)
            pltpu.sync_copy(as_hbm.at[pl.ds(kb, CH)], asb0)
            pltpu.sync_copy(ad_hbm.at[pl.ds(kb, CH)], adb0)
            for g in range(CH // LANES):
                o = g * LANES
                a = asb0[pl.ds(o, LANES)] + adb0[pl.ds(o, LANES)]
                wv0[pl.ds(o, LANES)] = jnp.exp(_leaky(a))
            _scale_rows(rows0, wv0)
            pltpu.sync_copy(rows0, sp_out.at[pl.ds(kb, CH)])
            pltpu.sync_copy(wv0, sp_dn.at[pl.ds(kb, CH)])

    def _init_zero(rbase, nchunks):
        _zero_rows(rows0)
        for k in range(nchunks):
            kb = rbase + k * CH
            pltpu.sync_copy(rows0, sp_out.at[pl.ds(kb, CH)])
            pltpu.sync_copy(zv, sp_dn.at[pl.ds(kb, CH)])

    @pl.when(jnp.logical_and(cid == 0, sid < NS - 1))
    def _():
        _init_self(sid * ROWS_MAIN, 8)

    @pl.when(jnp.logical_and(cid == 0, sid == NS - 1))
    def _():
        _init_self((NS - 1) * ROWS_MAIN, 5)

    @pl.when(jnp.logical_and(cid == 1, sid < NS - 1))
    def _():
        _init_zero(sid * ROWS_MAIN, 8)

    @pl.when(jnp.logical_and(cid == 1, sid == NS - 1))
    def _():
        _init_zero((NS - 1) * ROWS_MAIN, 5)

    plsc.subcore_barrier()

    def _issue(j, b, guard):
        pltpu.async_copy(as_hbm.at[sidx_all.at[pl.ds(j * CH, CH)]], asb[b], ga[b])
        pltpu.async_copy(ad_hbm.at[didx_all.at[j]], adb[b], gad[b])
        if guard:
            pltpu.make_async_copy(rows[b], sp_out.at[didx_all.at[j]],
                                  wrow[b]).wait()
            pltpu.make_async_copy(wv[b], sp_dn.at[didx_all.at[j]],
                                  wdn[b]).wait()
        pltpu.async_copy(h_hbm.at[sidx_all.at[pl.ds(j * CH, CH)]], rows[b], gr[b])

    def _process(j, b):
        pltpu.make_async_copy(as_hbm.at[sidx_all.at[pl.ds(j * CH, CH)]], asb[b], ga[b]).wait()
        pltpu.make_async_copy(ad_hbm.at[didx_all.at[j]], adb[b], gad[b]).wait()
        for g in range(CH // LANES):
            o = g * LANES
            a = asb[b][pl.ds(o, LANES)] + adb[b][pl.ds(o, LANES)]
            wv[b][pl.ds(o, LANES)] = jnp.exp(_leaky(a))
        pltpu.async_copy(wv[b], sp_dn.at[didx_all.at[j]], wdn[b], add=True)
        pltpu.make_async_copy(h_hbm.at[sidx_all.at[pl.ds(j * CH, CH)]], rows[b], gr[b]).wait()
        _scale_rows(rows[b], wv[b])
        pltpu.async_copy(rows[b], sp_out.at[didx_all.at[j]], wrow[b],
                         add=True)

    _issue(jnp.int32(0), 0, False)
    _issue(jnp.int32(1), 1, False)
    _process(jnp.int32(0), 0)
    _issue(jnp.int32(2), 0, True)
    _process(jnp.int32(1), 1)

    def _pair(ci, _):
        j = 2 + ci * 2
        _issue(j + 1, 1, True)
        _process(j, 0)
        _issue(j + 2, 0, True)
        _process(j + 1, 1)
        return 0

    lax.fori_loop(0, (NSUB - 3) // 2, _pair, 0)

    j_last = jnp.int32(NSUB - 1)
    _process(j_last, 0)

    pltpu.make_async_copy(rows[0], sp_out.at[didx_all.at[j_last]],
                          wrow[0]).wait()
    pltpu.make_async_copy(wv[0], sp_dn.at[didx_all.at[j_last]],
                          wdn[0]).wait()
    pltpu.make_async_copy(rows[1], sp_out.at[didx_all.at[j_last - 1]],
                          wrow[1]).wait()
    pltpu.make_async_copy(wv[1], sp_dn.at[didx_all.at[j_last - 1]],
                          wdn[1]).wait()

    plsc.subcore_barrier()

    def _export(rbase, nchunks):
        for k in range(nchunks):
            kb = rbase + k * CH
            pltpu.sync_copy(sp_out.at[pl.ds(kb, CH)], rows0)
            pltpu.sync_copy(rows0, out_hbm.at[cid, pl.ds(kb, CH), :])
        nr = nchunks * CH
        pltpu.sync_copy(sp_dn.at[pl.ds(rbase, nr)], rcomp_v.at[pl.ds(0, nr)])
        pltpu.sync_copy(rcomp_v.at[pl.ds(0, nr)],
                        dn_hbm.at[pl.ds(dnbase + rbase, nr)])

    @pl.when(sid < NS - 1)
    def _():
        _export(sid * ROWS_MAIN, 8)

    @pl.when(sid == NS - 1)
    def _():
        _export((NS - 1) * ROWS_MAIN, 5)


@functools.lru_cache(maxsize=None)
def _make_sc_edge():
    mesh = plsc.VectorSubcoreMesh(core_axis_name="c", subcore_axis_name="s",
                                  num_cores=NC, num_subcores=NS)
    scratch = [
        pltpu.VMEM((NSUB * CH,), jnp.int32),
        pltpu.VMEM((NSUB, CH), jnp.int32),
        pltpu.VMEM((ROWS_MAIN,), jnp.float32),
        pltpu.VMEM((CH,), jnp.float32),
        pltpu.VMEM((CH,), jnp.float32),
        pltpu.VMEM((CH,), jnp.float32),
        pltpu.VMEM((CH,), jnp.float32),
        pltpu.VMEM((CH,), jnp.float32),
        pltpu.VMEM((CH,), jnp.float32),
        pltpu.VMEM((CH, D), jnp.float32),
        pltpu.VMEM((CH, D), jnp.float32),
        pltpu.VMEM((CH,), jnp.float32),
        pltpu.SemaphoreType.DMA,
        pltpu.SemaphoreType.DMA,
        pltpu.SemaphoreType.DMA,
        pltpu.SemaphoreType.DMA,
        pltpu.SemaphoreType.DMA,
        pltpu.SemaphoreType.DMA,
        pltpu.SemaphoreType.DMA,
        pltpu.SemaphoreType.DMA,
        pltpu.SemaphoreType.DMA,
        pltpu.SemaphoreType.DMA,
        pltpu.VMEM_SHARED((N,), jnp.float32),
        pltpu.VMEM_SHARED((N, D), jnp.float32),
    ]
    return pl.kernel(
        _sc_edge_body,
        out_type=(jax.ShapeDtypeStruct((NC, N, D), jnp.float32),
                  jax.ShapeDtypeStruct((NC * N,), jnp.float32)),
        mesh=mesh,
        scratch_types=scratch,
        compiler_params=pltpu.CompilerParams(needs_layout_passes=False),
    )


def _tc_pre_body(x_ref, w_ref, as_ref, ad_ref, h_ref, asv_ref, adv_ref):
    h = jnp.dot(x_ref[...], w_ref[...], preferred_element_type=jnp.float32)
    h_ref[...] = h
    asv_ref[...] = jnp.sum(h * as_ref[...], axis=1).reshape(1, 1, -1)
    adv_ref[...] = jnp.sum(h * ad_ref[...], axis=1).reshape(1, 1, -1)


def _tc_pre(x, W, a_s, a_d):
    Cin, Cout = W.shape
    f = pl.pallas_call(
        _tc_pre_body,
        grid=(N // BR,),
        in_specs=[
            pl.BlockSpec((BR, Cin), lambda i: (i, 0)),
            pl.BlockSpec((Cin, Cout), lambda i: (0, 0)),
            pl.BlockSpec((1, Cout), lambda i: (0, 0)),
            pl.BlockSpec((1, Cout), lambda i: (0, 0)),
        ],
        out_specs=[
            pl.BlockSpec((BR, Cout), lambda i: (i, 0)),
            pl.BlockSpec((1, 1, BR), lambda i: (i, 0, 0)),
            pl.BlockSpec((1, 1, BR), lambda i: (i, 0, 0)),
        ],
        out_shape=[
            jax.ShapeDtypeStruct((N, Cout), jnp.float32),
            jax.ShapeDtypeStruct((N // BR, 1, BR), jnp.float32),
            jax.ShapeDtypeStruct((N // BR, 1, BR), jnp.float32),
        ],
    )
    return f(x, W, a_s, a_d)


def _tc_mid_body(p_ref, d0_ref, d1_ref, b_ref, w_ref, as_ref, ad_ref,
                 h_ref, asv_ref, adv_ref):
    d = d0_ref[0, 0, :] + d1_ref[0, 0, :]
    rec = 1.0 / (d + EPS)
    o = (p_ref[0] + p_ref[1]) * rec[:, None] + b_ref[...]
    o = jnp.maximum(o, 0.0)
    h = jnp.dot(o, w_ref[...], preferred_element_type=jnp.float32)
    h_ref[...] = h
    asv_ref[...] = jnp.sum(h * as_ref[...], axis=1).reshape(1, 1, -1)
    adv_ref[...] = jnp.sum(h * ad_ref[...], axis=1).reshape(1, 1, -1)


def _tc_mid(p, dn, b, W, a_s, a_d):
    Cin, Cout = W.shape
    d0 = dn[:N].reshape(N // BR, 1, BR)
    d1 = dn[N:].reshape(N // BR, 1, BR)
    f = pl.pallas_call(
        _tc_mid_body,
        grid=(N // BR,),
        in_specs=[
            pl.BlockSpec((NC, BR, Cin), lambda i: (0, i, 0)),
            pl.BlockSpec((1, 1, BR), lambda i: (i, 0, 0)),
            pl.BlockSpec((1, 1, BR), lambda i: (i, 0, 0)),
            pl.BlockSpec((1, Cin), lambda i: (0, 0)),
            pl.BlockSpec((Cin, Cout), lambda i: (0, 0)),
            pl.BlockSpec((1, Cout), lambda i: (0, 0)),
            pl.BlockSpec((1, Cout), lambda i: (0, 0)),
        ],
        out_specs=[
            pl.BlockSpec((BR, Cout), lambda i: (i, 0)),
            pl.BlockSpec((1, 1, BR), lambda i: (i, 0, 0)),
            pl.BlockSpec((1, 1, BR), lambda i: (i, 0, 0)),
        ],
        out_shape=[
            jax.ShapeDtypeStruct((N, Cout), jnp.float32),
            jax.ShapeDtypeStruct((N // BR, 1, BR), jnp.float32),
            jax.ShapeDtypeStruct((N // BR, 1, BR), jnp.float32),
        ],
    )
    return f(p, d0, d1, b.reshape(1, -1), W, a_s, a_d)


def _tc_fin_body(C, p_ref, d0_ref, d1_ref, b_ref, o_ref):
    d = d0_ref[0, 0, :] + d1_ref[0, 0, :]
    rec = 1.0 / (d + EPS)
    o_ref[...] = (p_ref[0, :, :C] + p_ref[1, :, :C]) * rec[:, None] + b_ref[...]


def _tc_fin(p, dn, b):
    C = b.size
    Cp = p.shape[-1]
    d0 = dn[:N].reshape(N // BR, 1, BR)
    d1 = dn[N:].reshape(N // BR, 1, BR)
    f = pl.pallas_call(
        functools.partial(_tc_fin_body, C),
        grid=(N // BR,),
        in_specs=[
            pl.BlockSpec((NC, BR, Cp), lambda i: (0, i, 0)),
            pl.BlockSpec((1, 1, BR), lambda i: (i, 0, 0)),
            pl.BlockSpec((1, 1, BR), lambda i: (i, 0, 0)),
            pl.BlockSpec((1, C), lambda i: (0, 0)),
        ],
        out_specs=pl.BlockSpec((BR, C), lambda i: (i, 0)),
        out_shape=jax.ShapeDtypeStruct((N, C), jnp.float32),
    )
    return f(p, d0, d1, b.reshape(1, -1))


def kernel(x, edge_index, W1, as1, ad1, b1, W2, as2, ad2, b2,
           W3, as3, ad3, b3):
    src2 = edge_index[0].astype(jnp.int32)
    dst2 = edge_index[1].astype(jnp.int32).reshape(NC * NS, NSUB, CH)

    sc = _make_sc_edge()

    pad = 128 - W3.shape[1]
    W3p = jnp.pad(W3, ((0, 0), (0, pad)))
    as3p = jnp.pad(as3, ((0, 0), (0, pad)))
    ad3p = jnp.pad(ad3, ((0, 0), (0, pad)))

    h, asv, adv = _tc_pre(x, W1, as1, ad1)
    p, dn = sc(h, asv.reshape(N), adv.reshape(N), src2, dst2)
    h, asv, adv = _tc_mid(p, dn, b1, W2, as2, ad2)
    p, dn = sc(h, asv.reshape(N), adv.reshape(N), src2, dst2)
    h, asv, adv = _tc_mid(p, dn, b2, W3p, as3p, ad3p)
    p, dn = sc(h, asv.reshape(N), adv.reshape(N), src2, dst2)
    return _tc_fin(p, dn, b3)


# speedup vs baseline: 59.0591x; 1.1272x over previous
import functools

import jax
import jax.numpy as jnp
from jax import lax
from jax.experimental import pallas as pl
from jax.experimental.pallas import tpu as pltpu
from jax.experimental.pallas import tpu_sc as plsc

N = 10000
E = 320000
NC = 2
NS = 16
LANES = 16
D = 128
CH = 80
NSUB = E // (NC * NS * CH)
ROWS_MAIN = 640
ROWS_LAST = N - (NS - 1) * ROWS_MAIN
NEG_SLOPE = 0.2
EPS = 1e-16
BR = 1000
KD = D // LANES


def _leaky(v):
    return jnp.where(v >= 0.0, v, NEG_SLOPE * v)


def _sc_edge_body(h_hbm, as_hbm, ad_hbm, src2_hbm, dst2_hbm,
                  out_hbm, dn_hbm,
                  sidx_all, didx_all, rcomp_v,
                  asb0, asb1, adb0, adb1, wv0, wv1, rows0, rows1, zv,
                  ga0, ga1, gad0, gad1, gr0, gr1, wdn0, wdn1, wrow0, wrow1,
                  sp_dn, sp_out):
    cid = lax.axis_index("c")
    sid = lax.axis_index("s")
    wid = sid * NC + cid
    dnbase = cid * N

    asb = [asb0, asb1]
    adb = [adb0, adb1]
    wv = [wv0, wv1]
    rows = [rows0, rows1]
    ga = [ga0, ga1]
    gad = [gad0, gad1]
    gr = [gr0, gr1]
    wdn = [wdn0, wdn1]
    wrow = [wrow0, wrow1]

    pltpu.sync_copy(src2_hbm.at[pl.ds(wid * NSUB * CH, NSUB * CH)], sidx_all)
    pltpu.sync_copy(dst2_hbm.at[wid], didx_all)

    for g in range(CH // LANES):
        zv[pl.ds(g * LANES, LANES)] = jnp.zeros((LANES,), jnp.float32)

    def _zero_rows(rref):
        def _z(j, _):
            for kk in range(KD):
                rref[j, pl.ds(kk * LANES, LANES)] = jnp.zeros(
                    (LANES,), jnp.float32)
            return 0
        lax.fori_loop(0, CH, _z, 0)

    def _scale_rows(rref, cref):
        def _s(g, _):
            c16 = cref[pl.ds(g * LANES, LANES)]
            for jj in range(LANES):
                c = c16[jj]
                row = g * LANES + jj
                for kk in range(KD):
                    rref[row, pl.ds(kk * LANES, LANES)] = (
                        rref[row, pl.ds(kk * LANES, LANES)] * c)
            return 0
        lax.fori_loop(0, CH // LANES, _s, 0)

    def _init_zero(rbase, nchunks):
        _zero_rows(rows0)
        for k in range(nchunks):
            kb = rbase + k * CH
            pltpu.sync_copy(rows0, sp_out.at[pl.ds(kb, CH)])
            pltpu.sync_copy(zv, sp_dn.at[pl.ds(kb, CH)])

    @pl.when(sid < NS - 1)
    def _():
        _init_zero(sid * ROWS_MAIN, 8)

    @pl.when(sid == NS - 1)
    def _():
        _init_zero((NS - 1) * ROWS_MAIN, 5)

    plsc.subcore_barrier()

    def _issue(j, b, guard):
        pltpu.async_copy(as_hbm.at[sidx_all.at[pl.ds(j * CH, CH)]], asb[b], ga[b])
        pltpu.async_copy(ad_hbm.at[didx_all.at[j]], adb[b], gad[b])
        if guard:
            pltpu.make_async_copy(rows[b], sp_out.at[didx_all.at[j]],
                                  wrow[b]).wait()
            pltpu.make_async_copy(wv[b], sp_dn.at[didx_all.at[j]],
                                  wdn[b]).wait()
        pltpu.async_copy(h_hbm.at[sidx_all.at[pl.ds(j * CH, CH)]], rows[b], gr[b])

    def _process(j, b):
        pltpu.make_async_copy(as_hbm.at[sidx_all.at[pl.ds(j * CH, CH)]], asb[b], ga[b]).wait()
        pltpu.make_async_copy(ad_hbm.at[didx_all.at[j]], adb[b], gad[b]).wait()
        for g in range(CH // LANES):
            o = g * LANES
            a = asb[b][pl.ds(o, LANES)] + adb[b][pl.ds(o, LANES)]
            wv[b][pl.ds(o, LANES)] = jnp.exp(_leaky(a))
        pltpu.async_copy(wv[b], sp_dn.at[didx_all.at[j]], wdn[b], add=True)
        pltpu.make_async_copy(h_hbm.at[sidx_all.at[pl.ds(j * CH, CH)]], rows[b], gr[b]).wait()
        _scale_rows(rows[b], wv[b])
        pltpu.async_copy(rows[b], sp_out.at[didx_all.at[j]], wrow[b],
                         add=True)

    _issue(jnp.int32(0), 0, False)
    _issue(jnp.int32(1), 1, False)
    _process(jnp.int32(0), 0)
    _issue(jnp.int32(2), 0, True)
    _process(jnp.int32(1), 1)

    def _pair(ci, _):
        j = 2 + ci * 2
        _issue(j + 1, 1, True)
        _process(j, 0)
        _issue(j + 2, 0, True)
        _process(j + 1, 1)
        return 0

    lax.fori_loop(0, (NSUB - 3) // 2, _pair, 0)

    j_last = jnp.int32(NSUB - 1)
    _process(j_last, 0)

    pltpu.make_async_copy(rows[0], sp_out.at[didx_all.at[j_last]],
                          wrow[0]).wait()
    pltpu.make_async_copy(wv[0], sp_dn.at[didx_all.at[j_last]],
                          wdn[0]).wait()
    pltpu.make_async_copy(rows[1], sp_out.at[didx_all.at[j_last - 1]],
                          wrow[1]).wait()
    pltpu.make_async_copy(wv[1], sp_dn.at[didx_all.at[j_last - 1]],
                          wdn[1]).wait()

    plsc.subcore_barrier()

    def _export(rbase, nchunks):
        nr = nchunks * CH
        pltpu.sync_copy(sp_out.at[pl.ds(rbase, nr)],
                        out_hbm.at[cid, pl.ds(rbase, nr), :])
        pltpu.sync_copy(sp_dn.at[pl.ds(rbase, nr)], rcomp_v.at[pl.ds(0, nr)])
        pltpu.sync_copy(rcomp_v.at[pl.ds(0, nr)],
                        dn_hbm.at[pl.ds(dnbase + rbase, nr)])

    @pl.when(sid < NS - 1)
    def _():
        _export(sid * ROWS_MAIN, 8)

    @pl.when(sid == NS - 1)
    def _():
        _export((NS - 1) * ROWS_MAIN, 5)


@functools.lru_cache(maxsize=None)
def _make_sc_edge():
    mesh = plsc.VectorSubcoreMesh(core_axis_name="c", subcore_axis_name="s",
                                  num_cores=NC, num_subcores=NS)
    scratch = [
        pltpu.VMEM((NSUB * CH,), jnp.int32),
        pltpu.VMEM((NSUB, CH), jnp.int32),
        pltpu.VMEM((ROWS_MAIN,), jnp.float32),
        pltpu.VMEM((CH,), jnp.float32),
        pltpu.VMEM((CH,), jnp.float32),
        pltpu.VMEM((CH,), jnp.float32),
        pltpu.VMEM((CH,), jnp.float32),
        pltpu.VMEM((CH,), jnp.float32),
        pltpu.VMEM((CH,), jnp.float32),
        pltpu.VMEM((CH, D), jnp.float32),
        pltpu.VMEM((CH, D), jnp.float32),
        pltpu.VMEM((CH,), jnp.float32),
        pltpu.SemaphoreType.DMA,
        pltpu.SemaphoreType.DMA,
        pltpu.SemaphoreType.DMA,
        pltpu.SemaphoreType.DMA,
        pltpu.SemaphoreType.DMA,
        pltpu.SemaphoreType.DMA,
        pltpu.SemaphoreType.DMA,
        pltpu.SemaphoreType.DMA,
        pltpu.SemaphoreType.DMA,
        pltpu.SemaphoreType.DMA,
        pltpu.VMEM_SHARED((N,), jnp.float32),
        pltpu.VMEM_SHARED((N, D), jnp.float32),
    ]
    return pl.kernel(
        _sc_edge_body,
        out_type=(jax.ShapeDtypeStruct((NC, N, D), jnp.float32),
                  jax.ShapeDtypeStruct((NC * N,), jnp.float32)),
        mesh=mesh,
        scratch_types=scratch,
        compiler_params=pltpu.CompilerParams(needs_layout_passes=False),
    )


def _tc_pre_body(x_ref, w_ref, as_ref, ad_ref, h_ref, asv_ref, adv_ref):
    h = jnp.dot(x_ref[...], w_ref[...], preferred_element_type=jnp.float32)
    h_ref[...] = h
    asv_ref[...] = jnp.sum(h * as_ref[...], axis=1).reshape(1, 1, -1)
    adv_ref[...] = jnp.sum(h * ad_ref[...], axis=1).reshape(1, 1, -1)


def _tc_pre(x, W, a_s, a_d):
    Cin, Cout = W.shape
    f = pl.pallas_call(
        _tc_pre_body,
        grid=(N // BR,),
        in_specs=[
            pl.BlockSpec((BR, Cin), lambda i: (i, 0)),
            pl.BlockSpec((Cin, Cout), lambda i: (0, 0)),
            pl.BlockSpec((1, Cout), lambda i: (0, 0)),
            pl.BlockSpec((1, Cout), lambda i: (0, 0)),
        ],
        out_specs=[
            pl.BlockSpec((BR, Cout), lambda i: (i, 0)),
            pl.BlockSpec((1, 1, BR), lambda i: (i, 0, 0)),
            pl.BlockSpec((1, 1, BR), lambda i: (i, 0, 0)),
        ],
        out_shape=[
            jax.ShapeDtypeStruct((N, Cout), jnp.float32),
            jax.ShapeDtypeStruct((N // BR, 1, BR), jnp.float32),
            jax.ShapeDtypeStruct((N // BR, 1, BR), jnp.float32),
        ],
    )
    return f(x, W, a_s, a_d)


def _tc_mid_body(p_ref, d0_ref, d1_ref, h_ref_in, asv_in, adv_in,
                 b_ref, w_ref, as_ref, ad_ref,
                 h_ref, asv_ref, adv_ref):
    wl = jnp.exp(_leaky(asv_in[0, 0, :] + adv_in[0, 0, :]))
    rec = 1.0 / (d0_ref[0, 0, :] + d1_ref[0, 0, :] + wl + EPS)
    o = ((p_ref[0] + p_ref[1] + wl[:, None] * h_ref_in[...])
         * rec[:, None] + b_ref[...])
    o = jnp.maximum(o, 0.0)
    h = jnp.dot(o, w_ref[...], preferred_element_type=jnp.float32)
    h_ref[...] = h
    asv_ref[...] = jnp.sum(h * as_ref[...], axis=1).reshape(1, 1, -1)
    adv_ref[...] = jnp.sum(h * ad_ref[...], axis=1).reshape(1, 1, -1)


def _tc_mid(p, dn, hprev, asv, adv, b, W, a_s, a_d):
    Cin, Cout = W.shape
    d0 = dn[:N].reshape(N // BR, 1, BR)
    d1 = dn[N:].reshape(N // BR, 1, BR)
    f = pl.pallas_call(
        _tc_mid_body,
        grid=(N // BR,),
        in_specs=[
            pl.BlockSpec((NC, BR, Cin), lambda i: (0, i, 0)),
            pl.BlockSpec((1, 1, BR), lambda i: (i, 0, 0)),
            pl.BlockSpec((1, 1, BR), lambda i: (i, 0, 0)),
            pl.BlockSpec((BR, Cin), lambda i: (i, 0)),
            pl.BlockSpec((1, 1, BR), lambda i: (i, 0, 0)),
            pl.BlockSpec((1, 1, BR), lambda i: (i, 0, 0)),
            pl.BlockSpec((1, Cin), lambda i: (0, 0)),
            pl.BlockSpec((Cin, Cout), lambda i: (0, 0)),
            pl.BlockSpec((1, Cout), lambda i: (0, 0)),
            pl.BlockSpec((1, Cout), lambda i: (0, 0)),
        ],
        out_specs=[
            pl.BlockSpec((BR, Cout), lambda i: (i, 0)),
            pl.BlockSpec((1, 1, BR), lambda i: (i, 0, 0)),
            pl.BlockSpec((1, 1, BR), lambda i: (i, 0, 0)),
        ],
        out_shape=[
            jax.ShapeDtypeStruct((N, Cout), jnp.float32),
            jax.ShapeDtypeStruct((N // BR, 1, BR), jnp.float32),
            jax.ShapeDtypeStruct((N // BR, 1, BR), jnp.float32),
        ],
    )
    return f(p, d0, d1, hprev, asv, adv, b.reshape(1, -1), W, a_s, a_d)


def _tc_fin_body(C, p_ref, d0_ref, d1_ref, h_ref_in, asv_in, adv_in,
                 b_ref, o_ref):
    wl = jnp.exp(_leaky(asv_in[0, 0, :] + adv_in[0, 0, :]))
    rec = 1.0 / (d0_ref[0, 0, :] + d1_ref[0, 0, :] + wl + EPS)
    o_ref[...] = ((p_ref[0, :, :C] + p_ref[1, :, :C]
                   + wl[:, None] * h_ref_in[:, :C])
                  * rec[:, None] + b_ref[...])


def _tc_fin(p, dn, hprev, asv, adv, b):
    C = b.size
    Cp = p.shape[-1]
    d0 = dn[:N].reshape(N // BR, 1, BR)
    d1 = dn[N:].reshape(N // BR, 1, BR)
    f = pl.pallas_call(
        functools.partial(_tc_fin_body, C),
        grid=(N // BR,),
        in_specs=[
            pl.BlockSpec((NC, BR, Cp), lambda i: (0, i, 0)),
            pl.BlockSpec((1, 1, BR), lambda i: (i, 0, 0)),
            pl.BlockSpec((1, 1, BR), lambda i: (i, 0, 0)),
            pl.BlockSpec((BR, Cp), lambda i: (i, 0)),
            pl.BlockSpec((1, 1, BR), lambda i: (i, 0, 0)),
            pl.BlockSpec((1, 1, BR), lambda i: (i, 0, 0)),
            pl.BlockSpec((1, C), lambda i: (0, 0)),
        ],
        out_specs=pl.BlockSpec((BR, C), lambda i: (i, 0)),
        out_shape=jax.ShapeDtypeStruct((N, C), jnp.float32),
    )
    return f(p, d0, d1, hprev, asv, adv, b.reshape(1, -1))


def kernel(x, edge_index, W1, as1, ad1, b1, W2, as2, ad2, b2,
           W3, as3, ad3, b3):
    src2 = edge_index[0].astype(jnp.int32)
    dst2 = edge_index[1].astype(jnp.int32).reshape(NC * NS, NSUB, CH)

    sc = _make_sc_edge()

    pad = 128 - W3.shape[1]
    W3p = jnp.pad(W3, ((0, 0), (0, pad)))
    as3p = jnp.pad(as3, ((0, 0), (0, pad)))
    ad3p = jnp.pad(ad3, ((0, 0), (0, pad)))

    h, asv, adv = _tc_pre(x, W1, as1, ad1)
    p, dn = sc(h, asv.reshape(N), adv.reshape(N), src2, dst2)
    h2, asv2, adv2 = _tc_mid(p, dn, h, asv, adv, b1, W2, as2, ad2)
    p, dn = sc(h2, asv2.reshape(N), adv2.reshape(N), src2, dst2)
    h3, asv3, adv3 = _tc_mid(p, dn, h2, asv2, adv2, b2, W3p, as3p, ad3p)
    p, dn = sc(h3, asv3.reshape(N), adv3.reshape(N), src2, dst2)
    return _tc_fin(p, dn, h3, asv3, adv3, b3)


# speedup vs baseline: 60.1072x; 1.0177x over previous
import functools

import jax
import jax.numpy as jnp
from jax import lax
from jax.experimental import pallas as pl
from jax.experimental.pallas import tpu as pltpu
from jax.experimental.pallas import tpu_sc as plsc

N = 10000
E = 320000
NC = 2
NS = 16
LANES = 16
D = 128
CH = 80
NSUB = E // (NC * NS * CH)
ROWS_MAIN = 640
ROWS_LAST = N - (NS - 1) * ROWS_MAIN
NEG_SLOPE = 0.2
EPS = 1e-16
BR = 1000
KD = D // LANES


def _leaky(v):
    return jnp.where(v >= 0.0, v, NEG_SLOPE * v)


def _sc_edge_body(h_hbm, as_hbm, ad_hbm, src2_hbm, dst2_hbm,
                  out_hbm, dn_hbm,
                  sidx_all, didx_all, rcomp_v,
                  asb0, asb1, adb0, adb1, wv0, wv1, rows0, rows1, zv,
                  ga0, ga1, gad0, gad1, gr0, gr1, wdn0, wdn1, wrow0, wrow1,
                  sp_dn, sp_out):
    cid = lax.axis_index("c")
    sid = lax.axis_index("s")
    wid = sid * NC + cid
    dnbase = cid * N

    asb = [asb0, asb1]
    adb = [adb0, adb1]
    wv = [wv0, wv1]
    rows = [rows0, rows1]
    ga = [ga0, ga1]
    gad = [gad0, gad1]
    gr = [gr0, gr1]
    wdn = [wdn0, wdn1]
    wrow = [wrow0, wrow1]

    cp_si = pltpu.async_copy(
        src2_hbm.at[pl.ds(wid * NSUB * CH, NSUB * CH)], sidx_all, ga0)
    cp_di = pltpu.async_copy(dst2_hbm.at[wid], didx_all, gad0)

    for g in range(CH // LANES):
        zv[pl.ds(g * LANES, LANES)] = jnp.zeros((LANES,), jnp.float32)

    def _zero_rows(rref):
        def _z(j, _):
            for kk in range(KD):
                rref[j, pl.ds(kk * LANES, LANES)] = jnp.zeros(
                    (LANES,), jnp.float32)
            return 0
        lax.fori_loop(0, CH, _z, 0)

    def _scale_rows(rref, cref):
        def _s(g, _):
            c16 = cref[pl.ds(g * LANES, LANES)]
            for jj in range(LANES):
                c = c16[jj]
                row = g * LANES + jj
                for kk in range(KD):
                    rref[row, pl.ds(kk * LANES, LANES)] = (
                        rref[row, pl.ds(kk * LANES, LANES)] * c)
            return 0
        lax.fori_loop(0, CH // LANES, _s, 0)

    def _init_zero(rbase, nchunks):
        _zero_rows(rows0)
        cps = []
        for k in range(nchunks):
            kb = rbase + k * CH
            cps.append(pltpu.async_copy(rows0, sp_out.at[pl.ds(kb, CH)], gr0))
            cps.append(pltpu.async_copy(zv, sp_dn.at[pl.ds(kb, CH)], gr1))
        for cp in cps:
            cp.wait()

    @pl.when(sid < NS - 1)
    def _():
        _init_zero(sid * ROWS_MAIN, 8)

    @pl.when(sid == NS - 1)
    def _():
        _init_zero((NS - 1) * ROWS_MAIN, 5)

    cp_si.wait()
    cp_di.wait()

    plsc.subcore_barrier()

    def _issue(j, b, guard):
        pltpu.async_copy(as_hbm.at[sidx_all.at[pl.ds(j * CH, CH)]], asb[b], ga[b])
        pltpu.async_copy(ad_hbm.at[didx_all.at[j]], adb[b], gad[b])
        if guard:
            pltpu.make_async_copy(rows[b], sp_out.at[didx_all.at[j]],
                                  wrow[b]).wait()
            pltpu.make_async_copy(wv[b], sp_dn.at[didx_all.at[j]],
                                  wdn[b]).wait()
        pltpu.async_copy(h_hbm.at[sidx_all.at[pl.ds(j * CH, CH)]], rows[b], gr[b])

    def _process(j, b):
        pltpu.make_async_copy(as_hbm.at[sidx_all.at[pl.ds(j * CH, CH)]], asb[b], ga[b]).wait()
        pltpu.make_async_copy(ad_hbm.at[didx_all.at[j]], adb[b], gad[b]).wait()
        for g in range(CH // LANES):
            o = g * LANES
            a = asb[b][pl.ds(o, LANES)] + adb[b][pl.ds(o, LANES)]
            wv[b][pl.ds(o, LANES)] = jnp.exp(_leaky(a))
        pltpu.async_copy(wv[b], sp_dn.at[didx_all.at[j]], wdn[b], add=True)
        pltpu.make_async_copy(h_hbm.at[sidx_all.at[pl.ds(j * CH, CH)]], rows[b], gr[b]).wait()
        _scale_rows(rows[b], wv[b])
        pltpu.async_copy(rows[b], sp_out.at[didx_all.at[j]], wrow[b],
                         add=True)

    _issue(jnp.int32(0), 0, False)
    _issue(jnp.int32(1), 1, False)
    _process(jnp.int32(0), 0)
    _issue(jnp.int32(2), 0, True)
    _process(jnp.int32(1), 1)

    def _pair(ci, _):
        j = 2 + ci * 2
        _issue(j + 1, 1, True)
        _process(j, 0)
        _issue(j + 2, 0, True)
        _process(j + 1, 1)
        return 0

    lax.fori_loop(0, (NSUB - 3) // 2, _pair, 0)

    j_last = jnp.int32(NSUB - 1)
    _process(j_last, 0)

    pltpu.make_async_copy(rows[0], sp_out.at[didx_all.at[j_last]],
                          wrow[0]).wait()
    pltpu.make_async_copy(wv[0], sp_dn.at[didx_all.at[j_last]],
                          wdn[0]).wait()
    pltpu.make_async_copy(rows[1], sp_out.at[didx_all.at[j_last - 1]],
                          wrow[1]).wait()
    pltpu.make_async_copy(wv[1], sp_dn.at[didx_all.at[j_last - 1]],
                          wdn[1]).wait()

    plsc.subcore_barrier()

    def _export(rbase, nchunks):
        nr = nchunks * CH
        pltpu.sync_copy(sp_out.at[pl.ds(rbase, nr)],
                        out_hbm.at[cid, pl.ds(rbase, nr), :])
        pltpu.sync_copy(sp_dn.at[pl.ds(rbase, nr)], rcomp_v.at[pl.ds(0, nr)])
        pltpu.sync_copy(rcomp_v.at[pl.ds(0, nr)],
                        dn_hbm.at[pl.ds(dnbase + rbase, nr)])

    @pl.when(sid < NS - 1)
    def _():
        _export(sid * ROWS_MAIN, 8)

    @pl.when(sid == NS - 1)
    def _():
        _export((NS - 1) * ROWS_MAIN, 5)


@functools.lru_cache(maxsize=None)
def _make_sc_edge():
    mesh = plsc.VectorSubcoreMesh(core_axis_name="c", subcore_axis_name="s",
                                  num_cores=NC, num_subcores=NS)
    scratch = [
        pltpu.VMEM((NSUB * CH,), jnp.int32),
        pltpu.VMEM((NSUB, CH), jnp.int32),
        pltpu.VMEM((ROWS_MAIN,), jnp.float32),
        pltpu.VMEM((CH,), jnp.float32),
        pltpu.VMEM((CH,), jnp.float32),
        pltpu.VMEM((CH,), jnp.float32),
        pltpu.VMEM((CH,), jnp.float32),
        pltpu.VMEM((CH,), jnp.float32),
        pltpu.VMEM((CH,), jnp.float32),
        pltpu.VMEM((CH, D), jnp.float32),
        pltpu.VMEM((CH, D), jnp.float32),
        pltpu.VMEM((CH,), jnp.float32),
        pltpu.SemaphoreType.DMA,
        pltpu.SemaphoreType.DMA,
        pltpu.SemaphoreType.DMA,
        pltpu.SemaphoreType.DMA,
        pltpu.SemaphoreType.DMA,
        pltpu.SemaphoreType.DMA,
        pltpu.SemaphoreType.DMA,
        pltpu.SemaphoreType.DMA,
        pltpu.SemaphoreType.DMA,
        pltpu.SemaphoreType.DMA,
        pltpu.VMEM_SHARED((N,), jnp.float32),
        pltpu.VMEM_SHARED((N, D), jnp.float32),
    ]
    return pl.kernel(
        _sc_edge_body,
        out_type=(jax.ShapeDtypeStruct((NC, N, D), jnp.float32),
                  jax.ShapeDtypeStruct((NC * N,), jnp.float32)),
        mesh=mesh,
        scratch_types=scratch,
        compiler_params=pltpu.CompilerParams(needs_layout_passes=False),
    )


def _tc_pre_body(x_ref, w_ref, as_ref, ad_ref, h_ref, asv_ref, adv_ref):
    h = jnp.dot(x_ref[...], w_ref[...], preferred_element_type=jnp.float32)
    h_ref[...] = h
    asv_ref[...] = jnp.sum(h * as_ref[...], axis=1).reshape(1, 1, -1)
    adv_ref[...] = jnp.sum(h * ad_ref[...], axis=1).reshape(1, 1, -1)


def _tc_pre(x, W, a_s, a_d):
    Cin, Cout = W.shape
    f = pl.pallas_call(
        _tc_pre_body,
        grid=(N // BR,),
        in_specs=[
            pl.BlockSpec((BR, Cin), lambda i: (i, 0)),
            pl.BlockSpec((Cin, Cout), lambda i: (0, 0)),
            pl.BlockSpec((1, Cout), lambda i: (0, 0)),
            pl.BlockSpec((1, Cout), lambda i: (0, 0)),
        ],
        out_specs=[
            pl.BlockSpec((BR, Cout), lambda i: (i, 0)),
            pl.BlockSpec((1, 1, BR), lambda i: (i, 0, 0)),
            pl.BlockSpec((1, 1, BR), lambda i: (i, 0, 0)),
        ],
        out_shape=[
            jax.ShapeDtypeStruct((N, Cout), jnp.float32),
            jax.ShapeDtypeStruct((N // BR, 1, BR), jnp.float32),
            jax.ShapeDtypeStruct((N // BR, 1, BR), jnp.float32),
        ],
    )
    return f(x, W, a_s, a_d)


def _tc_mid_body(p_ref, d0_ref, d1_ref, h_ref_in, asv_in, adv_in,
                 b_ref, w_ref, as_ref, ad_ref,
                 h_ref, asv_ref, adv_ref):
    wl = jnp.exp(_leaky(asv_in[0, 0, :] + adv_in[0, 0, :]))
    rec = 1.0 / (d0_ref[0, 0, :] + d1_ref[0, 0, :] + wl + EPS)
    o = ((p_ref[0] + p_ref[1] + wl[:, None] * h_ref_in[...])
         * rec[:, None] + b_ref[...])
    o = jnp.maximum(o, 0.0)
    h = jnp.dot(o, w_ref[...], preferred_element_type=jnp.float32)
    h_ref[...] = h
    asv_ref[...] = jnp.sum(h * as_ref[...], axis=1).reshape(1, 1, -1)
    adv_ref[...] = jnp.sum(h * ad_ref[...], axis=1).reshape(1, 1, -1)


def _tc_mid(p, dn, hprev, asv, adv, b, W, a_s, a_d):
    Cin, Cout = W.shape
    d0 = dn[:N].reshape(N // BR, 1, BR)
    d1 = dn[N:].reshape(N // BR, 1, BR)
    f = pl.pallas_call(
        _tc_mid_body,
        grid=(N // BR,),
        in_specs=[
            pl.BlockSpec((NC, BR, Cin), lambda i: (0, i, 0)),
            pl.BlockSpec((1, 1, BR), lambda i: (i, 0, 0)),
            pl.BlockSpec((1, 1, BR), lambda i: (i, 0, 0)),
            pl.BlockSpec((BR, Cin), lambda i: (i, 0)),
            pl.BlockSpec((1, 1, BR), lambda i: (i, 0, 0)),
            pl.BlockSpec((1, 1, BR), lambda i: (i, 0, 0)),
            pl.BlockSpec((1, Cin), lambda i: (0, 0)),
            pl.BlockSpec((Cin, Cout), lambda i: (0, 0)),
            pl.BlockSpec((1, Cout), lambda i: (0, 0)),
            pl.BlockSpec((1, Cout), lambda i: (0, 0)),
        ],
        out_specs=[
            pl.BlockSpec((BR, Cout), lambda i: (i, 0)),
            pl.BlockSpec((1, 1, BR), lambda i: (i, 0, 0)),
            pl.BlockSpec((1, 1, BR), lambda i: (i, 0, 0)),
        ],
        out_shape=[
            jax.ShapeDtypeStruct((N, Cout), jnp.float32),
            jax.ShapeDtypeStruct((N // BR, 1, BR), jnp.float32),
            jax.ShapeDtypeStruct((N // BR, 1, BR), jnp.float32),
        ],
    )
    return f(p, d0, d1, hprev, asv, adv, b.reshape(1, -1), W, a_s, a_d)


def _tc_fin_body(C, p_ref, d0_ref, d1_ref, h_ref_in, asv_in, adv_in,
                 b_ref, o_ref):
    wl = jnp.exp(_leaky(asv_in[0, 0, :] + adv_in[0, 0, :]))
    rec = 1.0 / (d0_ref[0, 0, :] + d1_ref[0, 0, :] + wl + EPS)
    o_ref[...] = ((p_ref[0, :, :C] + p_ref[1, :, :C]
                   + wl[:, None] * h_ref_in[:, :C])
                  * rec[:, None] + b_ref[...])


def _tc_fin(p, dn, hprev, asv, adv, b):
    C = b.size
    Cp = p.shape[-1]
    d0 = dn[:N].reshape(N // BR, 1, BR)
    d1 = dn[N:].reshape(N // BR, 1, BR)
    f = pl.pallas_call(
        functools.partial(_tc_fin_body, C),
        grid=(N // BR,),
        in_specs=[
            pl.BlockSpec((NC, BR, Cp), lambda i: (0, i, 0)),
            pl.BlockSpec((1, 1, BR), lambda i: (i, 0, 0)),
            pl.BlockSpec((1, 1, BR), lambda i: (i, 0, 0)),
            pl.BlockSpec((BR, Cp), lambda i: (i, 0)),
            pl.BlockSpec((1, 1, BR), lambda i: (i, 0, 0)),
            pl.BlockSpec((1, 1, BR), lambda i: (i, 0, 0)),
            pl.BlockSpec((1, C), lambda i: (0, 0)),
        ],
        out_specs=pl.BlockSpec((BR, C), lambda i: (i, 0)),
        out_shape=jax.ShapeDtypeStruct((N, C), jnp.float32),
    )
    return f(p, d0, d1, hprev, asv, adv, b.reshape(1, -1))


def kernel(x, edge_index, W1, as1, ad1, b1, W2, as2, ad2, b2,
           W3, as3, ad3, b3):
    src2 = edge_index[0].astype(jnp.int32)
    dst2 = edge_index[1].astype(jnp.int32).reshape(NC * NS, NSUB, CH)

    sc = _make_sc_edge()

    pad = 128 - W3.shape[1]
    W3p = jnp.pad(W3, ((0, 0), (0, pad)))
    as3p = jnp.pad(as3, ((0, 0), (0, pad)))
    ad3p = jnp.pad(ad3, ((0, 0), (0, pad)))

    h, asv, adv = _tc_pre(x, W1, as1, ad1)
    p, dn = sc(h, asv.reshape(N), adv.reshape(N), src2, dst2)
    h2, asv2, adv2 = _tc_mid(p, dn, h, asv, adv, b1, W2, as2, ad2)
    p, dn = sc(h2, asv2.reshape(N), adv2.reshape(N), src2, dst2)
    h3, asv3, adv3 = _tc_mid(p, dn, h2, asv2, adv2, b2, W3p, as3p, ad3p)
    p, dn = sc(h3, asv3.reshape(N), adv3.reshape(N), src2, dst2)
    return _tc_fin(p, dn, h3, asv3, adv3, b3)
